# baseline (speedup 1.0000x reference)
import jax
import jax.numpy as jnp
from jax.experimental import pallas as pl
from jax.experimental.pallas import tpu as pltpu

_DIM = 64
_Q = 16
_PACK = 4
_W = _DIM * _PACK
_EPS = 1e-8


def _row_to_col(v, zero):
    eye = (jax.lax.broadcasted_iota(jnp.int32, (_Q, _Q), 0)
           == jax.lax.broadcasted_iota(jnp.int32, (_Q, _Q), 1))
    b = jnp.broadcast_to(v, (_Q, _Q))
    return jnp.sum(jnp.where(eye, b, zero), axis=1, keepdims=True)


def _scan_body(q_ref, x_ref, w_ref, b_ref, out_ref,
               bsim_ref, bmem_ref, wq_ref, wn_ref):
    i = pl.program_id(0)
    nblk = pl.num_programs(0)
    g = x_ref.shape[0]

    @pl.when(i == 0)
    def _init():
        bsim_ref[...] = jnp.full_like(bsim_ref, -jnp.inf)
        bmem_ref[...] = jnp.zeros_like(bmem_ref)
        q = q_ref[...]
        qn = q / (jnp.sqrt(jnp.sum(q * q, axis=1, keepdims=True)) + _EPS)
        qnt = qn.T
        cols = []
        for j in range(_PACK):
            cols.append(jnp.pad(qnt, ((_DIM * j, _W - _DIM * (j + 1)), (0, 0))))
        wq_ref[...] = jnp.concatenate(cols, axis=1)
        r = jax.lax.broadcasted_iota(jnp.int32, (_W, _PACK * _Q), 0)
        c = jax.lax.broadcasted_iota(jnp.int32, (_W, _PACK * _Q), 1)
        wn_ref[...] = ((r // _DIM) == (c // _Q)).astype(jnp.float32)

    x = x_ref[...]
    s = jax.lax.dot_general(x, wq_ref[...], (((1,), (0,)), ((), ())),
                            preferred_element_type=jnp.float32)
    t = jax.lax.dot_general(x * x, wn_ref[...], (((1,), (0,)), ((), ())),
                            preferred_element_type=jnp.float32)
    sims = s * (1.0 / (jnp.sqrt(t) + _EPS))

    cmax = jnp.max(sims, axis=0, keepdims=True)
    rowi = jax.lax.broadcasted_iota(jnp.int32, sims.shape, 0)
    rmin = jnp.min(jnp.where(sims >= cmax, rowi, g), axis=0, keepdims=True)
    coli = jax.lax.broadcasted_iota(jnp.int32, (1, _PACK * _Q), 1)
    lidx = _PACK * rmin + coli // _Q

    bs = cmax[:, 0:_Q]
    bg = lidx[:, 0:_Q]
    for j in range(1, _PACK):
        sj = cmax[:, _Q * j:_Q * (j + 1)]
        gj = lidx[:, _Q * j:_Q * (j + 1)]
        better = (sj > bs) | ((sj == bs) & (gj < bg))
        bs = jnp.where(better, sj, bs)
        bg = jnp.where(better, gj, bg)
    bs_col = _row_to_col(bs, jnp.float32(0))
    bg_col = _row_to_col(bg, jnp.int32(0))

    g_star = bg_col // _PACK
    j_star = bg_col % _PACK
    oh = (jax.lax.broadcasted_iota(jnp.int32, (_Q, g), 1) == g_star)
    r4 = jax.lax.dot_general(oh.astype(jnp.float32), x, (((1,), (0,)), ((), ())),
                             preferred_element_type=jnp.float32)
    bm = jnp.zeros((_Q, _DIM), jnp.float32)
    for j in range(_PACK):
        bm = jnp.where(j_star == j, r4[:, _DIM * j:_DIM * (j + 1)], bm)

    upd = bs_col > bsim_ref[...]
    bsim_ref[...] = jnp.where(upd, bs_col, bsim_ref[...])
    bmem_ref[...] = jnp.where(upd, bm, bmem_ref[...])

    @pl.when(i == nblk - 1)
    def _final():
        o = jax.lax.dot_general(bmem_ref[...], w_ref[...], (((1,), (1,)), ((), ())),
                                preferred_element_type=jnp.float32)
        o = o + b_ref[...]
        gate = (bsim_ref[...] > 0.6).astype(jnp.float32)
        out_ref[...] = o * gate


def kernel(query, memories, W_dec, b_dec):
    cap = memories.shape[0]
    x = memories.reshape(cap // _PACK, _W)
    gblk = 10000
    grid = (cap // _PACK) // gblk
    b2 = b_dec.reshape(1, _DIM)

    out = pl.pallas_call(
        _scan_body,
        grid=(grid,),
        in_specs=[
            pl.BlockSpec((_Q, _DIM), lambda i: (0, 0)),
            pl.BlockSpec((gblk, _W), lambda i: (i, 0)),
            pl.BlockSpec((_DIM, _DIM), lambda i: (0, 0)),
            pl.BlockSpec((1, _DIM), lambda i: (0, 0)),
        ],
        out_specs=pl.BlockSpec((_Q, _DIM), lambda i: (0, 0)),
        out_shape=jax.ShapeDtypeStruct((_Q, _DIM), jnp.float32),
        scratch_shapes=[
            pltpu.VMEM((_Q, 1), jnp.float32),
            pltpu.VMEM((_Q, _DIM), jnp.float32),
            pltpu.VMEM((_W, _PACK * _Q), jnp.float32),
            pltpu.VMEM((_W, _PACK * _Q), jnp.float32),
        ],
        compiler_params=pltpu.CompilerParams(
            dimension_semantics=("arbitrary",),
        ),
    )(query, x, W_dec, b2)
    return out


# speedup vs baseline: 1.5143x; 1.5143x over previous
import jax
import jax.numpy as jnp
from jax.experimental import pallas as pl
from jax.experimental.pallas import tpu as pltpu

_DIM = 64
_Q = 16
_EPS = 1e-8


def _scan_body(q_ref, x_ref, mem_ref, w_ref, b_ref, out_ref,
               bsim_ref, bidx_ref, gbuf_ref, sem):
    i = pl.program_id(0)
    nblk = pl.num_programs(0)
    blk = x_ref.shape[0]

    @pl.when(i == 0)
    def _init():
        bsim_ref[...] = jnp.full_like(bsim_ref, -jnp.inf)
        bidx_ref[...] = jnp.zeros_like(bidx_ref)

    q = q_ref[...]
    qn = q / (jnp.sqrt(jnp.sum(q * q, axis=1, keepdims=True)) + _EPS)

    x = x_ref[...]
    s = jax.lax.dot_general(qn, x, (((1,), (1,)), ((), ())),
                            preferred_element_type=jnp.float32)
    ones = jnp.ones((1, _DIM), jnp.float32)
    t = jax.lax.dot_general(ones, x * x, (((1,), (1,)), ((), ())),
                            preferred_element_type=jnp.float32)
    sims = s * (1.0 / (jnp.sqrt(t) + _EPS))

    bmax = jnp.max(sims, axis=1, keepdims=True)
    col = jax.lax.broadcasted_iota(jnp.int32, sims.shape, 1)
    lidx = jnp.min(jnp.where(sims >= bmax, col, blk), axis=1, keepdims=True)

    upd = bmax > bsim_ref[...]
    bsim_ref[...] = jnp.where(upd, bmax, bsim_ref[...])
    bidx_ref[...] = jnp.where(upd, i * blk + lidx, bidx_ref[...])

    @pl.when(i == nblk - 1)
    def _final():
        bidx = bidx_ref[...]
        rowq = jax.lax.broadcasted_iota(jnp.int32, (_Q, 1), 0)
        for qi in range(_Q):
            idx = jnp.sum(jnp.where(rowq == qi, bidx, 0))
            cp = pltpu.make_async_copy(
                mem_ref.at[pl.ds(idx, 1), :], gbuf_ref.at[pl.ds(qi, 1), :], sem)
            cp.start()
            cp.wait()
        bm = gbuf_ref[...]
        o = jax.lax.dot_general(bm, w_ref[...], (((1,), (1,)), ((), ())),
                                preferred_element_type=jnp.float32)
        o = o + b_ref[...]
        gate = (bsim_ref[...] > 0.6).astype(jnp.float32)
        out_ref[...] = o * gate


def kernel(query, memories, W_dec, b_dec):
    cap = memories.shape[0]
    blk = 20000
    grid = cap // blk
    b2 = b_dec.reshape(1, _DIM)

    out = pl.pallas_call(
        _scan_body,
        grid=(grid,),
        in_specs=[
            pl.BlockSpec((_Q, _DIM), lambda i: (0, 0)),
            pl.BlockSpec((blk, _DIM), lambda i: (i, 0)),
            pl.BlockSpec(memory_space=pl.ANY),
            pl.BlockSpec((_DIM, _DIM), lambda i: (0, 0)),
            pl.BlockSpec((1, _DIM), lambda i: (0, 0)),
        ],
        out_specs=pl.BlockSpec((_Q, _DIM), lambda i: (0, 0)),
        out_shape=jax.ShapeDtypeStruct((_Q, _DIM), jnp.float32),
        scratch_shapes=[
            pltpu.VMEM((_Q, 1), jnp.float32),
            pltpu.VMEM((_Q, 1), jnp.int32),
            pltpu.VMEM((_Q, _DIM), jnp.float32),
            pltpu.SemaphoreType.DMA,
        ],
        compiler_params=pltpu.CompilerParams(
            dimension_semantics=("arbitrary",),
        ),
    )(query, memories, memories, W_dec, b2)
    return out
